# baseline (speedup 1.0000x reference)
import functools

import jax
import jax.numpy as jnp
from jax import lax
from jax.experimental import pallas as pl
from jax.experimental.pallas import tpu as pltpu
from jax.experimental.pallas import tpu_sc as plsc

_VOCAB = 1000000
_EMBED = 64
_SEQ = 200
_BATCH = 4096

_VCOLS = 16384
_TCG = (_VOCAB + _VCOLS - 1) // _VCOLS


def _tc_matvec_body(e_ref, w_ref, o_ref):
    o_ref[...] = jnp.sum(e_ref[...] * w_ref[...], axis=0)


def _tc_matvec(embed_table, W):
    et = embed_table.T
    wt = W.reshape(_EMBED, 1)
    return pl.pallas_call(
        _tc_matvec_body,
        grid=(_TCG,),
        in_specs=[
            pl.BlockSpec((_EMBED, _VCOLS), lambda i: (0, i)),
            pl.BlockSpec((_EMBED, 1), lambda i: (0, 0)),
        ],
        out_specs=pl.BlockSpec((_VCOLS,), lambda i: (i,)),
        out_shape=jax.ShapeDtypeStruct((_VOCAB,), jnp.float32),
    )(et, wt)


_NC = 2
_NS = 16
_NW = _NC * _NS
_CB = _BATCH // _NW


def _sc_pool(x, p, lengths, b16):
    mesh = plsc.VectorSubcoreMesh(core_axis_name="c", subcore_axis_name="s")

    @functools.partial(
        pl.kernel,
        out_type=jax.ShapeDtypeStruct((_BATCH,), jnp.float32),
        mesh=mesh,
        scratch_types=[
            pltpu.VMEM((_SEQ, _CB), jnp.int32),
            pltpu.VMEM((_SEQ, _CB), jnp.float32),
            pltpu.VMEM((_CB,), jnp.int32),
            pltpu.VMEM((16,), jnp.float32),
            pltpu.VMEM((_CB,), jnp.float32),
            pltpu.SemaphoreType.DMA,
        ],
    )
    def k(x_hbm, p_hbm, len_hbm, b_hbm, out_hbm, xv, gv, lenv, bv, outv, sem):
        wid = lax.axis_index("s") * _NC + lax.axis_index("c")
        base = wid * _CB
        pltpu.sync_copy(x_hbm.at[:, pl.ds(base, _CB)], xv)
        pltpu.sync_copy(len_hbm.at[pl.ds(base, _CB)], lenv)
        pltpu.sync_copy(b_hbm, bv)

        def fire(l, carry):
            pltpu.async_copy(p_hbm.at[xv.at[l]], gv.at[l], sem)
            return carry

        lax.fori_loop(0, _SEQ, fire, 0)

        def drain(l, carry):
            pltpu.make_async_copy(p_hbm.at[xv.at[0]], gv.at[0], sem).wait()
            return carry

        lax.fori_loop(0, _SEQ, drain, 0)
        bias = bv[...]
        for j in range(_CB // 16):
            sl = pl.ds(j * 16, 16)

            def body(l, acc):
                return acc + gv[l, sl]

            acc = lax.fori_loop(0, _SEQ, body, jnp.zeros((16,), jnp.float32))
            lf = lenv[sl].astype(jnp.float32)
            z = acc / lf + bias
            outv[sl] = 1.0 / (1.0 + jnp.exp(-z))
        pltpu.sync_copy(outv, out_hbm.at[pl.ds(base, _CB)])

    return k(x, p, lengths, b16)


def kernel(x, lengths, embed_table, W, b):
    p = _tc_matvec(embed_table, W.astype(jnp.float32))
    b16 = jnp.broadcast_to(b.astype(jnp.float32), (16,))
    return _sc_pool(x.astype(jnp.int32), p, lengths.astype(jnp.int32), b16)


# speedup vs baseline: 1.0947x; 1.0947x over previous
import functools

import jax
import jax.numpy as jnp
from jax import lax
from jax.experimental import pallas as pl
from jax.experimental.pallas import tpu as pltpu
from jax.experimental.pallas import tpu_sc as plsc

_VOCAB = 1000000
_EMBED = 64
_SEQ = 200
_BATCH = 4096

_VCOLS = 49152
_TCG = (_VOCAB + _VCOLS - 1) // _VCOLS


def _tc_matvec_body(e_ref, w_ref, o_ref):
    o_ref[...] = jnp.sum(e_ref[...] * w_ref[...], axis=0)


def _tc_matvec(embed_table, W):
    et = embed_table.T
    wt = W.reshape(_EMBED, 1)
    return pl.pallas_call(
        _tc_matvec_body,
        grid=(_TCG,),
        in_specs=[
            pl.BlockSpec((_EMBED, _VCOLS), lambda i: (0, i)),
            pl.BlockSpec((_EMBED, 1), lambda i: (0, 0)),
        ],
        out_specs=pl.BlockSpec((_VCOLS,), lambda i: (i,)),
        out_shape=jax.ShapeDtypeStruct((_VOCAB,), jnp.float32),
    )(et, wt)


_NC = 2
_NS = 16
_NW = _NC * _NS
_CB = _BATCH // _NW


def _sc_pool(x, p, lengths, b16):
    mesh = plsc.VectorSubcoreMesh(core_axis_name="c", subcore_axis_name="s")

    @functools.partial(
        pl.kernel,
        out_type=jax.ShapeDtypeStruct((_BATCH,), jnp.float32),
        mesh=mesh,
        scratch_types=[
            pltpu.VMEM((_SEQ, _CB), jnp.int32),
            pltpu.VMEM((_SEQ, _CB), jnp.float32),
            pltpu.VMEM((_CB,), jnp.int32),
            pltpu.VMEM((16,), jnp.float32),
            pltpu.VMEM((_CB,), jnp.float32),
            pltpu.SemaphoreType.DMA,
        ],
    )
    def k(x_hbm, p_hbm, len_hbm, b_hbm, out_hbm, xv, gv, lenv, bv, outv, sem):
        wid = lax.axis_index("s") * _NC + lax.axis_index("c")
        base = wid * _CB
        pltpu.sync_copy(x_hbm.at[:, pl.ds(base, _CB)], xv)
        pltpu.sync_copy(len_hbm.at[pl.ds(base, _CB)], lenv)
        pltpu.sync_copy(b_hbm, bv)

        def fire(l, carry):
            pltpu.async_copy(p_hbm.at[xv.at[l]], gv.at[l], sem)
            return carry

        lax.fori_loop(0, _SEQ, fire, 0)

        def drain(l, carry):
            pltpu.make_async_copy(p_hbm.at[xv.at[0]], gv.at[0], sem).wait()
            return carry

        lax.fori_loop(0, _SEQ, drain, 0)
        bias = bv[...]
        for j in range(_CB // 16):
            sl = pl.ds(j * 16, 16)

            def body(l, acc):
                return acc + gv[l, sl]

            acc = lax.fori_loop(0, _SEQ, body, jnp.zeros((16,), jnp.float32))
            lf = lenv[sl].astype(jnp.float32)
            z = acc / lf + bias
            outv[sl] = 1.0 / (1.0 + jnp.exp(-z))
        pltpu.sync_copy(outv, out_hbm.at[pl.ds(base, _CB)])

    return k(x, p, lengths, b16)


def kernel(x, lengths, embed_table, W, b):
    p = _tc_matvec(embed_table, W.astype(jnp.float32))
    b16 = jnp.broadcast_to(b.astype(jnp.float32), (16,))
    return _sc_pool(x.astype(jnp.int32), p, lengths.astype(jnp.int32), b16)


# speedup vs baseline: 1.1332x; 1.0352x over previous
import functools

import jax
import jax.numpy as jnp
from jax import lax
from jax.experimental import pallas as pl
from jax.experimental.pallas import tpu as pltpu
from jax.experimental.pallas import tpu_sc as plsc

_VOCAB = 1000000
_EMBED = 64
_SEQ = 200
_BATCH = 4096

_VCOLS = 49152
_TCG = (_VOCAB + _VCOLS - 1) // _VCOLS


def _tc_matvec_body(e_ref, w_ref, o_ref):
    o_ref[...] = jnp.sum(e_ref[...] * w_ref[...], axis=0)


def _tc_matvec(embed_table, W):
    et = embed_table.T
    wt = W.reshape(_EMBED, 1)
    return pl.pallas_call(
        _tc_matvec_body,
        grid=(_TCG,),
        in_specs=[
            pl.BlockSpec((_EMBED, _VCOLS), lambda i: (0, i)),
            pl.BlockSpec((_EMBED, 1), lambda i: (0, 0)),
        ],
        out_specs=pl.BlockSpec((_VCOLS,), lambda i: (i,)),
        out_shape=jax.ShapeDtypeStruct((_VOCAB,), jnp.float32),
    )(et, wt)


_NC = 2
_NS = 16
_NW = _NC * _NS
_CB = _BATCH // _NW


def _sc_pool(x, p, lengths, b16):
    mesh = plsc.VectorSubcoreMesh(core_axis_name="c", subcore_axis_name="s")

    @functools.partial(
        pl.kernel,
        out_type=jax.ShapeDtypeStruct((_BATCH,), jnp.float32),
        mesh=mesh,
        scratch_types=[
            pltpu.VMEM((_SEQ, _CB), jnp.int32),
            pltpu.VMEM((_SEQ, _CB), jnp.float32),
            pltpu.VMEM((_CB,), jnp.int32),
            pltpu.VMEM((16,), jnp.float32),
            pltpu.VMEM((_CB,), jnp.float32),
            pltpu.SemaphoreType.DMA,
            pltpu.SemaphoreType.DMA,
        ],
    )
    def k(x_hbm, p_hbm, len_hbm, b_hbm, out_hbm, xv, gv, lenv, bv, outv,
          sem, sem2):
        wid = lax.axis_index("s") * _NC + lax.axis_index("c")
        base = wid * _CB
        pltpu.sync_copy(x_hbm.at[:, pl.ds(base, _CB)], xv)
        pltpu.sync_copy(len_hbm.at[pl.ds(base, _CB)], lenv)
        pltpu.sync_copy(b_hbm, bv)

        half = _SEQ // 2

        def fire(l, carry):
            pltpu.async_copy(p_hbm.at[xv.at[l]], gv.at[l], sem)
            return carry

        def fire2(l, carry):
            pltpu.async_copy(p_hbm.at[xv.at[l]], gv.at[l], sem2)
            return carry

        lax.fori_loop(0, half, fire, 0)
        lax.fori_loop(half, _SEQ, fire2, 0)

        def drain(l, carry):
            pltpu.make_async_copy(p_hbm.at[xv.at[0]], gv.at[0], sem).wait()
            return carry

        def drain2(l, carry):
            pltpu.make_async_copy(p_hbm.at[xv.at[0]], gv.at[0], sem2).wait()
            return carry

        def acc_rows(lo, hi, j, init):
            sl = pl.ds(j * 16, 16)

            def body(l4, acc):
                l = lo + l4 * 4
                return acc + gv[l, sl] + gv[l + 1, sl] \
                    + gv[l + 2, sl] + gv[l + 3, sl]

            return lax.fori_loop(0, (hi - lo) // 4, body, init)

        lax.fori_loop(0, half, drain, 0)
        partial = [acc_rows(0, half, j, jnp.zeros((16,), jnp.float32))
                   for j in range(_CB // 16)]
        lax.fori_loop(half, _SEQ, drain2, 0)
        bias = bv[...]
        for j in range(_CB // 16):
            sl = pl.ds(j * 16, 16)
            acc = acc_rows(half, _SEQ, j, partial[j])
            lf = lenv[sl].astype(jnp.float32)
            z = acc / lf + bias
            outv[sl] = 1.0 / (1.0 + jnp.exp(-z))
        pltpu.sync_copy(outv, out_hbm.at[pl.ds(base, _CB)])

    return k(x, p, lengths, b16)


def kernel(x, lengths, embed_table, W, b):
    p = _tc_matvec(embed_table, W.astype(jnp.float32))
    b16 = jnp.broadcast_to(b.astype(jnp.float32), (16,))
    return _sc_pool(x.astype(jnp.int32), p, lengths.astype(jnp.int32), b16)
